# baseline (speedup 1.0000x reference)
import functools

import jax
import jax.numpy as jnp
from jax import lax
from jax.experimental import pallas as pl
from jax.experimental.pallas import tpu as pltpu
from jax.experimental.pallas import tpu_sc as plsc

N = 10000
E = 320000
IN_C = 128
HID = 128
OUT_C = 64

NC = 2
NS = 16
NW = NC * NS
EPW = E // NW
EPWP = 10240
NCHP = 80
CHP = 128
CHG = 64
NCH = EPWP // CHG
NP = 10240
NPAD = 10240
SLC = NP // NS
RPS = NPAD // NS
PADIDX = NPAD - 1


def _mesh():
    return plsc.VectorSubcoreMesh(
        core_axis_name="c", subcore_axis_name="s",
        num_cores=NC, num_subcores=NS)


def _unpack_to(pk_v, row, half, lo_ref, lo_slot, hi_ref, hi_slot):
    for k in range(CHG // 16):
        v = pk_v[row, pl.ds(64 * half + 16 * k, 16)]
        if lo_ref is not None:
            lo_ref[lo_slot, pl.ds(16 * k, 16)] = v & 0xFFFF
        if hi_ref is not None:
            hi_ref[hi_slot, pl.ds(16 * k, 16)] = lax.shift_right_logical(v, 16)


def _deg_call(pk_pos, zeros_np):
    @functools.partial(
        pl.kernel,
        out_type=jax.ShapeDtypeStruct((NC, NP), jnp.float32),
        mesh=_mesh(),
        scratch_types=[
            pltpu.VMEM((NCHP, CHP), jnp.int32),
            pltpu.VMEM((CHP,), jnp.int32),
            pltpu.VMEM((CHP,), jnp.float32),
            pltpu.VMEM_SHARED((NP,), jnp.float32),
        ],
    )
    def deg_k(pk_hbm, z_hbm, out_hbm, pk_v, di_v, ones_v, tab_sh):
        c = lax.axis_index("c")
        s = lax.axis_index("s")
        wid = s * NC + c
        for k in range(CHP // 16):
            ones_v[pl.ds(k * 16, 16)] = jnp.ones((16,), jnp.float32)
        pltpu.sync_copy(z_hbm.at[pl.ds(s * SLC, SLC)],
                        tab_sh.at[pl.ds(s * SLC, SLC)])
        pltpu.sync_copy(pk_hbm.at[wid], pk_v)
        plsc.subcore_barrier()

        def body(j, carry):
            for k in range(CHP // 16):
                v = pk_v[j, pl.ds(16 * k, 16)]
                di_v[pl.ds(16 * k, 16)] = lax.shift_right_logical(v, 16)
            pltpu.sync_copy(ones_v, tab_sh.at[di_v], add=True)
            return carry

        lax.fori_loop(0, NCHP, body, 0)
        plsc.subcore_barrier()
        pltpu.sync_copy(tab_sh.at[pl.ds(s * SLC, SLC)],
                        out_hbm.at[c, pl.ds(s * SLC, SLC)])

    return deg_k(pk_pos, zeros_np)


def _agg_call(u, pk_pos, zeros_tab):
    width = u.shape[1]

    @functools.partial(
        pl.kernel,
        out_type=jax.ShapeDtypeStruct((NC, NPAD, width), jnp.float32),
        mesh=_mesh(),
        scratch_types=[
            pltpu.VMEM((NCHP, CHP), jnp.int32),
            pltpu.VMEM((2, CHP), jnp.int32),
            pltpu.VMEM((2, CHP), jnp.int32),
            pltpu.VMEM((2, CHP, width), jnp.float32),
            pltpu.VMEM_SHARED((NPAD, width), jnp.float32),
            pltpu.SemaphoreType.DMA,
        ],
    )
    def agg_k(u_hbm, pk_hbm, z_hbm, out_hbm,
              pk_v, sic, dic, gb_v, acc_sh, sem):
        c = lax.axis_index("c")
        s = lax.axis_index("s")
        wid = s * NC + c
        pltpu.sync_copy(z_hbm.at[pl.ds(s * RPS, RPS)],
                        acc_sh.at[pl.ds(s * RPS, RPS)])
        pltpu.sync_copy(pk_hbm.at[wid], pk_v)
        plsc.subcore_barrier()

        def unpack(row, slot):
            for k in range(CHP // 16):
                v = pk_v[row, pl.ds(16 * k, 16)]
                sic[slot, pl.ds(16 * k, 16)] = v & 0xFFFF
                dic[slot, pl.ds(16 * k, 16)] = lax.shift_right_logical(v, 16)

        def body(t, carry):
            unpack(t, 0)
            pltpu.async_copy(u_hbm.at[sic.at[0]], gb_v.at[0], sem).wait()
            pltpu.sync_copy(gb_v.at[0], acc_sh.at[dic.at[0]], add=True)
            return carry

        lax.fori_loop(0, NCHP, body, 0)
        plsc.subcore_barrier()
        pltpu.sync_copy(acc_sh.at[pl.ds(s * RPS, RPS)],
                        out_hbm.at[c, pl.ds(s * RPS, RPS)])

    return agg_k(u, pk_pos, zeros_tab)


def _decode_call(z, pk_pos, pk_neg):
    D = z.shape[1]

    @functools.partial(
        pl.kernel,
        out_type=(jax.ShapeDtypeStruct((NW, EPWP), jnp.float32),
                  jax.ShapeDtypeStruct((NW, EPWP), jnp.float32)),
        compiler_params=pltpu.CompilerParams(needs_layout_passes=False),
        mesh=_mesh(),
        scratch_types=[
            pltpu.VMEM((NCHP, CHP), jnp.int32),
            pltpu.VMEM((2 * CHG,), jnp.int32),
            pltpu.VMEM((2 * CHG,), jnp.int32),
            pltpu.VMEM((2 * CHG, HID), jnp.float32),
            pltpu.VMEM((2 * CHG, HID), jnp.float32),
            pltpu.VMEM((CHG,), jnp.float32),
            pltpu.VMEM((16 * 17,), jnp.float32),
            pltpu.SemaphoreType.DMA,
            pltpu.SemaphoreType.DMA,
        ],
    )
    def dec_k(z_hbm, pkp_h, pkn_h, po_h, no_h,
              pk_v, iac, ibc, a_v, b_v, sc_v, tp_v, sa, sb):
        c = lax.axis_index("c")
        s = lax.axis_index("s")
        wid = s * NC + c
        rows16 = lax.iota(jnp.int32, 16)

        def unpack_to(row, half, lo_ref, lo_slot, hi_ref, hi_slot):
            rvec = jnp.full((16,), row, jnp.int32)
            for k in range(CHG // 16):
                cvec = rows16 + (64 * half + 16 * k)
                v = plsc.load_gather(pk_v, [rvec, cvec])
                lo_ref[pl.ds(lo_slot * CHG + 16 * k, 16)] = v & 0xFFFF
                hi_ref[pl.ds(hi_slot * CHG + 16 * k, 16)] = (
                    lax.shift_right_logical(v, 16))

        def compute_chunk(bt):
            for q in range(CHG // 16):
                for l in range(16):
                    e = bt * CHG + 16 * q + l
                    part = (a_v[e, pl.ds(0, 16)] *
                            b_v[e, pl.ds(0, 16)])
                    for k in range(1, OUT_C // 16):
                        part = part + (a_v[e, pl.ds(16 * k, 16)] *
                                       b_v[e, pl.ds(16 * k, 16)])
                    tp_v[pl.ds(17 * l, 16)] = part
                res = plsc.load_gather(tp_v, [rows16 * 17])
                for k in range(1, 16):
                    res = res + plsc.load_gather(tp_v, [rows16 * 17 + k])
                sc_v[pl.ds(16 * q, 16)] = res

        def do_list(pk_hbm, o_h):
            pltpu.sync_copy(pk_hbm.at[wid], pk_v)
            unpack_to(0, 0, iac, 0, ibc, 0)

            def islc(ref, bb):
                return ref.at[pl.ds(bb * CHG, CHG)]

            pltpu.async_copy(z_hbm.at[islc(iac, 0)], islc(a_v, 0), sa)
            pltpu.async_copy(z_hbm.at[islc(ibc, 0)], islc(b_v, 0), sb)
            unpack_to(0, 1, iac, 1, ibc, 1)

            def step(t, bt):
                pltpu.make_async_copy(
                    z_hbm.at[islc(iac, bt)], islc(a_v, bt), sa).wait()
                pltpu.make_async_copy(
                    z_hbm.at[islc(ibc, bt)], islc(b_v, bt), sb).wait()

                @pl.when(t + 1 < NCH)
                def _():
                    pltpu.async_copy(
                        z_hbm.at[islc(iac, 1 - bt)], islc(a_v, 1 - bt), sa)
                    pltpu.async_copy(
                        z_hbm.at[islc(ibc, 1 - bt)], islc(b_v, 1 - bt), sb)

                compute_chunk(bt)
                pltpu.sync_copy(sc_v, o_h.at[wid, pl.ds(t * CHG, CHG)])

                @pl.when(t + 2 < NCH)
                def _():
                    unpack_to((t + 2) // 2, bt, iac, bt, ibc, bt)

            def body(p, carry):
                step(2 * p, 0)
                step(2 * p + 1, 1)
                return carry

            lax.fori_loop(0, NCH // 2, body, 0)

        do_list(pkp_h, po_h)
        do_list(pkn_h, no_h)

    return dec_k(z, pk_pos, pk_neg)


def _dis_rows(dg_ref):
    deg = dg_ref[:, 0] + dg_ref[:, 1] + 1.0
    return lax.rsqrt(jnp.maximum(deg, 1e-12))


def _matmul(a, w_ref):
    return lax.dot_general(a, w_ref[...], (((1,), (0,)), ((), ())),
                           precision=lax.Precision.HIGHEST,
                           preferred_element_type=jnp.float32)


_BR = 2048


def _pre_call(degp, x, w1):
    def body(dg_ref, x_ref, w_ref, o_ref):
        dis = _dis_rows(dg_ref)[:, None]
        o_ref[...] = dis * _matmul(x_ref[...], w_ref)

    return pl.pallas_call(
        body,
        grid=(NPAD // _BR,),
        in_specs=[
            pl.BlockSpec((_BR, NC), lambda i: (i, 0)),
            pl.BlockSpec((_BR, IN_C), lambda i: (i, 0)),
            pl.BlockSpec((IN_C, HID), lambda i: (0, 0)),
        ],
        out_specs=pl.BlockSpec((_BR, HID), lambda i: (i, 0)),
        out_shape=jax.ShapeDtypeStruct((NPAD, HID), jnp.float32),
    )(degp, x, w1)


def _mid_call(degp, sp, u_prev, b, w_next):
    width = sp.shape[2]
    h = w_next.shape[1]

    def body(dg_ref, sp_ref, u_ref, b_ref, w_ref, o_ref):
        dis = _dis_rows(dg_ref)[:, None]
        stot = sp_ref[0] + sp_ref[1] + u_ref[...]
        zz = jnp.maximum(stot * dis + b_ref[...], 0.0)
        o_ref[...] = dis * _matmul(zz, w_ref)

    return pl.pallas_call(
        body,
        grid=(NPAD // _BR,),
        in_specs=[
            pl.BlockSpec((_BR, NC), lambda i: (i, 0)),
            pl.BlockSpec((NC, _BR, width), lambda i: (0, i, 0)),
            pl.BlockSpec((_BR, width), lambda i: (i, 0)),
            pl.BlockSpec((1, width), lambda i: (0, 0)),
            pl.BlockSpec((width, h), lambda i: (0, 0)),
        ],
        out_specs=pl.BlockSpec((_BR, h), lambda i: (i, 0)),
        out_shape=jax.ShapeDtypeStruct((NPAD, h), jnp.float32),
    )(degp, sp, u_prev, b, w_next)


def _fin_call(degp, sp, u_prev, b):
    width = sp.shape[2]

    def body(dg_ref, sp_ref, u_ref, b_ref, o_ref):
        dis = _dis_rows(dg_ref)[:, None]
        stot = sp_ref[0] + sp_ref[1] + u_ref[...]
        o_ref[...] = stot * dis + b_ref[...]

    return pl.pallas_call(
        body,
        grid=(NPAD // _BR,),
        in_specs=[
            pl.BlockSpec((_BR, NC), lambda i: (i, 0)),
            pl.BlockSpec((NC, _BR, width), lambda i: (0, i, 0)),
            pl.BlockSpec((_BR, width), lambda i: (i, 0)),
            pl.BlockSpec((1, width), lambda i: (0, 0)),
        ],
        out_specs=pl.BlockSpec((_BR, width), lambda i: (i, 0)),
        out_shape=jax.ShapeDtypeStruct((NPAD, width), jnp.float32),
    )(degp, sp, u_prev, b)


def _pack_edges(rows, cols):
    r2 = rows.reshape(NW, EPW)
    c2 = cols.reshape(NW, EPW)
    r2 = jnp.pad(r2, ((0, 0), (0, EPWP - EPW)), constant_values=PADIDX)
    c2 = jnp.pad(c2, ((0, 0), (0, EPWP - EPW)), constant_values=PADIDX)
    return (r2 + c2 * 65536).reshape(NW, NCHP, CHP)


def kernel(x, pos_edge_index, neg_edge_index, W1, b1, W2, b2, W3, b3):
    pk_pos = _pack_edges(pos_edge_index[0], pos_edge_index[1])
    pk_neg = _pack_edges(neg_edge_index[0], neg_edge_index[1])

    zeros_np = jnp.zeros((NP,), jnp.float32)
    zeros128 = jnp.zeros((NPAD, HID), jnp.float32)
    xp = jnp.pad(x, ((0, NPAD - N), (0, 0)))
    w3p = jnp.pad(W3, ((0, 0), (0, HID - OUT_C)))
    b3p = jnp.pad(b3, (0, HID - OUT_C))

    degp = _deg_call(pk_pos, zeros_np).T
    u1 = _pre_call(degp, xp, W1)
    s1 = _agg_call(u1, pk_pos, zeros128)
    u2 = _mid_call(degp, s1, u1, b1.reshape(1, HID), W2)
    s2 = _agg_call(u2, pk_pos, zeros128)
    u3 = _mid_call(degp, s2, u2, b2.reshape(1, HID), w3p)
    s3 = _agg_call(u3, pk_pos, zeros128)
    z3 = _fin_call(degp, s3, u3, b3p.reshape(1, HID))
    pos_p, neg_p = _decode_call(z3, pk_pos, pk_neg)
    pos_score = pos_p[:, :EPW].reshape(E)
    neg_score = neg_p[:, :EPW].reshape(E)
    return (pos_score, neg_score)


# speedup vs baseline: 2.1862x; 2.1862x over previous
import functools

import jax
import jax.numpy as jnp
from jax import lax
from jax.experimental import pallas as pl
from jax.experimental.pallas import tpu as pltpu
from jax.experimental.pallas import tpu_sc as plsc

N = 10000
E = 320000
IN_C = 128
HID = 128
OUT_C = 64

NC = 2
NS = 16
NW = NC * NS
EPW = E // NW
EPWP = 10240
NCHP = 80
CHP = 128
CHG = 64
NCH = EPWP // CHG
NP = 10240
NPAD = 10240
SLC = NP // NS
RPS = NPAD // NS
PADIDX = NPAD - 1


def _mesh():
    return plsc.VectorSubcoreMesh(
        core_axis_name="c", subcore_axis_name="s",
        num_cores=NC, num_subcores=NS)


def _unpack_to(pk_v, row, half, lo_ref, lo_slot, hi_ref, hi_slot):
    for k in range(CHG // 16):
        v = pk_v[row, pl.ds(64 * half + 16 * k, 16)]
        if lo_ref is not None:
            lo_ref[lo_slot, pl.ds(16 * k, 16)] = v & 0xFFFF
        if hi_ref is not None:
            hi_ref[hi_slot, pl.ds(16 * k, 16)] = lax.shift_right_logical(v, 16)


def _deg_call(pk_pos, zeros_np):
    @functools.partial(
        pl.kernel,
        out_type=jax.ShapeDtypeStruct((NC, NP), jnp.float32),
        mesh=_mesh(),
        scratch_types=[
            pltpu.VMEM((NCHP, CHP), jnp.int32),
            pltpu.VMEM((CHP,), jnp.int32),
            pltpu.VMEM((CHP,), jnp.float32),
            pltpu.VMEM_SHARED((NP,), jnp.float32),
        ],
    )
    def deg_k(pk_hbm, z_hbm, out_hbm, pk_v, di_v, ones_v, tab_sh):
        c = lax.axis_index("c")
        s = lax.axis_index("s")
        wid = s * NC + c
        for k in range(CHP // 16):
            ones_v[pl.ds(k * 16, 16)] = jnp.ones((16,), jnp.float32)
        pltpu.sync_copy(z_hbm.at[pl.ds(s * SLC, SLC)],
                        tab_sh.at[pl.ds(s * SLC, SLC)])
        pltpu.sync_copy(pk_hbm.at[wid], pk_v)
        plsc.subcore_barrier()

        def body(j, carry):
            for k in range(CHP // 16):
                v = pk_v[j, pl.ds(16 * k, 16)]
                di_v[pl.ds(16 * k, 16)] = lax.shift_right_logical(v, 16)
            pltpu.sync_copy(ones_v, tab_sh.at[di_v], add=True)
            return carry

        lax.fori_loop(0, NCHP, body, 0)
        plsc.subcore_barrier()
        pltpu.sync_copy(tab_sh.at[pl.ds(s * SLC, SLC)],
                        out_hbm.at[c, pl.ds(s * SLC, SLC)])

    return deg_k(pk_pos, zeros_np)


def _agg_call(u, pk_pos, zeros_tab):
    width = u.shape[1]

    @functools.partial(
        pl.kernel,
        out_type=jax.ShapeDtypeStruct((NC, NPAD, width), jnp.float32),
        mesh=_mesh(),
        scratch_types=[
            pltpu.VMEM((NCHP, CHP), jnp.int32),
            pltpu.VMEM((2, CHP), jnp.int32),
            pltpu.VMEM((2, CHP), jnp.int32),
            pltpu.VMEM((2, CHP, width), jnp.float32),
            pltpu.VMEM_SHARED((NPAD, width), jnp.float32),
            pltpu.SemaphoreType.DMA,
        ],
    )
    def agg_k(u_hbm, pk_hbm, z_hbm, out_hbm,
              pk_v, sic, dic, gb_v, acc_sh, sem):
        c = lax.axis_index("c")
        s = lax.axis_index("s")
        wid = s * NC + c
        pltpu.sync_copy(z_hbm.at[pl.ds(s * RPS, RPS)],
                        acc_sh.at[pl.ds(s * RPS, RPS)])
        pltpu.sync_copy(pk_hbm.at[wid], pk_v)
        plsc.subcore_barrier()

        def unpack(row, slot):
            for k in range(CHP // 16):
                v = pk_v[row, pl.ds(16 * k, 16)]
                sic[slot, pl.ds(16 * k, 16)] = v & 0xFFFF
                dic[slot, pl.ds(16 * k, 16)] = lax.shift_right_logical(v, 16)

        def body(t, carry):
            unpack(t, 0)
            pltpu.async_copy(u_hbm.at[sic.at[0]], gb_v.at[0], sem).wait()
            pltpu.sync_copy(gb_v.at[0], acc_sh.at[dic.at[0]], add=True)
            return carry

        lax.fori_loop(0, NCHP, body, 0)
        plsc.subcore_barrier()
        pltpu.sync_copy(acc_sh.at[pl.ds(s * RPS, RPS)],
                        out_hbm.at[c, pl.ds(s * RPS, RPS)])

    return agg_k(u, pk_pos, zeros_tab)


def _decode_call(z, pk_pos, pk_neg):
    D = z.shape[1]

    @functools.partial(
        pl.kernel,
        out_type=(jax.ShapeDtypeStruct((NW, EPWP), jnp.float32),
                  jax.ShapeDtypeStruct((NW, EPWP), jnp.float32)),
        compiler_params=pltpu.CompilerParams(needs_layout_passes=False),
        mesh=_mesh(),
        scratch_types=[
            pltpu.VMEM((NCHP, CHP), jnp.int32),
            pltpu.VMEM((2 * CHG,), jnp.int32),
            pltpu.VMEM((2 * CHG,), jnp.int32),
            pltpu.VMEM((2 * CHG, HID), jnp.float32),
            pltpu.VMEM((2 * CHG, HID), jnp.float32),
            pltpu.VMEM((CHG,), jnp.float32),
            pltpu.VMEM((16 * 17,), jnp.float32),
            pltpu.SemaphoreType.DMA,
            pltpu.SemaphoreType.DMA,
        ],
    )
    def dec_k(z_hbm, pkp_h, pkn_h, po_h, no_h,
              pk_v, iac, ibc, a_v, b_v, sc_v, tp_v, sa, sb):
        c = lax.axis_index("c")
        s = lax.axis_index("s")
        wid = s * NC + c
        rows16 = lax.iota(jnp.int32, 16)

        def unpack_to(row, half, lo_ref, lo_slot, hi_ref, hi_slot):
            rvec = jnp.full((16,), row, jnp.int32)
            for k in range(CHG // 16):
                cvec = rows16 + (64 * half + 16 * k)
                v = plsc.load_gather(pk_v, [rvec, cvec])
                lo_ref[pl.ds(lo_slot * CHG + 16 * k, 16)] = v & 0xFFFF
                hi_ref[pl.ds(hi_slot * CHG + 16 * k, 16)] = (
                    lax.shift_right_logical(v, 16))

        def compute_chunk(bt):
            for q in range(CHG // 16):
                for l in range(16):
                    e = bt * CHG + 16 * q + l
                    part = (a_v[e, pl.ds(0, 16)] *
                            b_v[e, pl.ds(0, 16)])
                    for k in range(1, OUT_C // 16):
                        part = part + (a_v[e, pl.ds(16 * k, 16)] *
                                       b_v[e, pl.ds(16 * k, 16)])
                    tp_v[pl.ds(17 * l, 16)] = part
                res = plsc.load_gather(tp_v, [rows16 * 17])
                for k in range(1, 16):
                    res = res + plsc.load_gather(tp_v, [rows16 * 17 + k])
                sc_v[pl.ds(16 * q, 16)] = res

        def do_list(pk_hbm, o_h):
            pltpu.sync_copy(pk_hbm.at[wid], pk_v)

            def islc(ref, bb):
                return ref.at[pl.ds(bb * CHG, CHG)]

            def step(p, h):
                t = 2 * p + h
                unpack_to(p, h, iac, 0, ibc, 0)
                da = pltpu.async_copy(
                    z_hbm.at[islc(iac, 0)], islc(a_v, 0), sa)
                db = pltpu.async_copy(
                    z_hbm.at[islc(ibc, 0)], islc(b_v, 0), sb)
                da.wait()
                db.wait()
                compute_chunk(0)
                pltpu.sync_copy(sc_v, o_h.at[wid, pl.ds(t * CHG, CHG)])

            def body(p, carry):
                step(p, 0)
                step(p, 1)
                return carry

            lax.fori_loop(0, NCH // 2, body, 0)

        do_list(pkp_h, po_h)
        do_list(pkn_h, no_h)

    return dec_k(z, pk_pos, pk_neg)


def _dis_rows(dg_ref):
    deg = dg_ref[:, 0] + dg_ref[:, 1] + 1.0
    return lax.rsqrt(jnp.maximum(deg, 1e-12))


def _matmul(a, w_ref):
    return lax.dot_general(a, w_ref[...], (((1,), (0,)), ((), ())),
                           precision=lax.Precision.HIGHEST,
                           preferred_element_type=jnp.float32)


_BR = 2048


def _pre_call(degp, x, w1):
    def body(dg_ref, x_ref, w_ref, o_ref):
        dis = _dis_rows(dg_ref)[:, None]
        o_ref[...] = dis * _matmul(x_ref[...], w_ref)

    return pl.pallas_call(
        body,
        grid=(NPAD // _BR,),
        in_specs=[
            pl.BlockSpec((_BR, NC), lambda i: (i, 0)),
            pl.BlockSpec((_BR, IN_C), lambda i: (i, 0)),
            pl.BlockSpec((IN_C, HID), lambda i: (0, 0)),
        ],
        out_specs=pl.BlockSpec((_BR, HID), lambda i: (i, 0)),
        out_shape=jax.ShapeDtypeStruct((NPAD, HID), jnp.float32),
    )(degp, x, w1)


def _mid_call(degp, sp, u_prev, b, w_next):
    width = sp.shape[2]
    h = w_next.shape[1]

    def body(dg_ref, sp_ref, u_ref, b_ref, w_ref, o_ref):
        dis = _dis_rows(dg_ref)[:, None]
        stot = sp_ref[0] + sp_ref[1] + u_ref[...]
        zz = jnp.maximum(stot * dis + b_ref[...], 0.0)
        o_ref[...] = dis * _matmul(zz, w_ref)

    return pl.pallas_call(
        body,
        grid=(NPAD // _BR,),
        in_specs=[
            pl.BlockSpec((_BR, NC), lambda i: (i, 0)),
            pl.BlockSpec((NC, _BR, width), lambda i: (0, i, 0)),
            pl.BlockSpec((_BR, width), lambda i: (i, 0)),
            pl.BlockSpec((1, width), lambda i: (0, 0)),
            pl.BlockSpec((width, h), lambda i: (0, 0)),
        ],
        out_specs=pl.BlockSpec((_BR, h), lambda i: (i, 0)),
        out_shape=jax.ShapeDtypeStruct((NPAD, h), jnp.float32),
    )(degp, sp, u_prev, b, w_next)


def _fin_call(degp, sp, u_prev, b):
    width = sp.shape[2]

    def body(dg_ref, sp_ref, u_ref, b_ref, o_ref):
        dis = _dis_rows(dg_ref)[:, None]
        stot = sp_ref[0] + sp_ref[1] + u_ref[...]
        o_ref[...] = stot * dis + b_ref[...]

    return pl.pallas_call(
        body,
        grid=(NPAD // _BR,),
        in_specs=[
            pl.BlockSpec((_BR, NC), lambda i: (i, 0)),
            pl.BlockSpec((NC, _BR, width), lambda i: (0, i, 0)),
            pl.BlockSpec((_BR, width), lambda i: (i, 0)),
            pl.BlockSpec((1, width), lambda i: (0, 0)),
        ],
        out_specs=pl.BlockSpec((_BR, width), lambda i: (i, 0)),
        out_shape=jax.ShapeDtypeStruct((NPAD, width), jnp.float32),
    )(degp, sp, u_prev, b)


def _pack_edges(rows, cols):
    pad = N + (jnp.arange(EPWP - EPW, dtype=jnp.int32) % (NPAD - N))
    pad = jnp.broadcast_to(pad, (NW, EPWP - EPW))
    r2 = jnp.concatenate([rows.reshape(NW, EPW), pad], axis=1)
    c2 = jnp.concatenate([cols.reshape(NW, EPW), pad], axis=1)
    return (r2 + c2 * 65536).reshape(NW, NCHP, CHP)


def kernel(x, pos_edge_index, neg_edge_index, W1, b1, W2, b2, W3, b3):
    pk_pos = _pack_edges(pos_edge_index[0], pos_edge_index[1])
    pk_neg = _pack_edges(neg_edge_index[0], neg_edge_index[1])

    zeros_np = jnp.zeros((NP,), jnp.float32)
    zeros128 = jnp.zeros((NPAD, HID), jnp.float32)
    xp = jnp.pad(x, ((0, NPAD - N), (0, 0)))
    w3p = jnp.pad(W3, ((0, 0), (0, HID - OUT_C)))
    b3p = jnp.pad(b3, (0, HID - OUT_C))

    degp = _deg_call(pk_pos, zeros_np).T
    u1 = _pre_call(degp, xp, W1)
    s1 = _agg_call(u1, pk_pos, zeros128)
    u2 = _mid_call(degp, s1, u1, b1.reshape(1, HID), W2)
    s2 = _agg_call(u2, pk_pos, zeros128)
    u3 = _mid_call(degp, s2, u2, b2.reshape(1, HID), w3p)
    s3 = _agg_call(u3, pk_pos, zeros128)
    z3 = _fin_call(degp, s3, u3, b3p.reshape(1, HID))
    pos_p, neg_p = _decode_call(z3, pk_pos, pk_neg)
    pos_score = pos_p[:, :EPW].reshape(E)
    neg_score = neg_p[:, :EPW].reshape(E)
    return (pos_score, neg_score)


# speedup vs baseline: 3.4324x; 1.5700x over previous
import functools

import jax
import jax.numpy as jnp
from jax import lax
from jax.experimental import pallas as pl
from jax.experimental.pallas import tpu as pltpu
from jax.experimental.pallas import tpu_sc as plsc

N = 10000
E = 320000
IN_C = 128
HID = 128
OUT_C = 64

NC = 2
NS = 16
NW = NC * NS
EPW = E // NW
EPWP = 10240
NCHP = 80
CHP = 128
CHG = 64
NCH = EPWP // CHG
NP = 10240
NPAD = 10240
SLC = NP // NS
RPS = NPAD // NS
PADIDX = NPAD - 1


def _mesh():
    return plsc.VectorSubcoreMesh(
        core_axis_name="c", subcore_axis_name="s",
        num_cores=NC, num_subcores=NS)


def _unpack_to(pk_v, row, half, lo_ref, lo_slot, hi_ref, hi_slot):
    for k in range(CHG // 16):
        v = pk_v[row, pl.ds(64 * half + 16 * k, 16)]
        if lo_ref is not None:
            lo_ref[lo_slot, pl.ds(16 * k, 16)] = v & 0xFFFF
        if hi_ref is not None:
            hi_ref[hi_slot, pl.ds(16 * k, 16)] = lax.shift_right_logical(v, 16)


def _deg_call(pk_pos, zeros_np):
    @functools.partial(
        pl.kernel,
        out_type=jax.ShapeDtypeStruct((NC, NP), jnp.float32),
        mesh=_mesh(),
        scratch_types=[
            pltpu.VMEM((NCHP, CHP), jnp.int32),
            pltpu.VMEM((CHP,), jnp.int32),
            pltpu.VMEM((CHP,), jnp.float32),
            pltpu.VMEM_SHARED((NP,), jnp.float32),
        ],
    )
    def deg_k(pk_hbm, z_hbm, out_hbm, pk_v, di_v, ones_v, tab_sh):
        c = lax.axis_index("c")
        s = lax.axis_index("s")
        wid = s * NC + c
        for k in range(CHP // 16):
            ones_v[pl.ds(k * 16, 16)] = jnp.ones((16,), jnp.float32)
        pltpu.sync_copy(z_hbm.at[pl.ds(s * SLC, SLC)],
                        tab_sh.at[pl.ds(s * SLC, SLC)])
        pltpu.sync_copy(pk_hbm.at[wid], pk_v)
        plsc.subcore_barrier()

        def body(j, carry):
            for k in range(CHP // 16):
                v = pk_v[j, pl.ds(16 * k, 16)]
                di_v[pl.ds(16 * k, 16)] = lax.shift_right_logical(v, 16)
            pltpu.sync_copy(ones_v, tab_sh.at[di_v], add=True)
            return carry

        lax.fori_loop(0, NCHP, body, 0)
        plsc.subcore_barrier()
        pltpu.sync_copy(tab_sh.at[pl.ds(s * SLC, SLC)],
                        out_hbm.at[c, pl.ds(s * SLC, SLC)])

    return deg_k(pk_pos, zeros_np)


def _agg_call(u, pk_pos, zeros_tab):
    width = u.shape[1]

    @functools.partial(
        pl.kernel,
        out_type=jax.ShapeDtypeStruct((NC, NPAD, width), jnp.float32),
        mesh=_mesh(),
        scratch_types=[
            pltpu.VMEM((NCHP, CHP), jnp.int32),
            pltpu.VMEM((2, CHP), jnp.int32),
            pltpu.VMEM((2, CHP), jnp.int32),
            pltpu.VMEM((2, CHP, width), jnp.float32),
            pltpu.VMEM_SHARED((NPAD, width), jnp.float32),
            pltpu.SemaphoreType.DMA,
            pltpu.SemaphoreType.DMA,
        ],
    )
    def agg_k(u_hbm, pk_hbm, z_hbm, out_hbm,
              pk_v, sic, dic, gb_v, acc_sh, sem, ssem):
        c = lax.axis_index("c")
        s = lax.axis_index("s")
        wid = s * NC + c
        pltpu.sync_copy(z_hbm.at[pl.ds(s * RPS, RPS)],
                        acc_sh.at[pl.ds(s * RPS, RPS)])
        pltpu.sync_copy(pk_hbm.at[wid], pk_v)
        plsc.subcore_barrier()

        def unpack(row, slot):
            for k in range(CHP // 16):
                v = pk_v[row, pl.ds(16 * k, 16)]
                sic[slot, pl.ds(16 * k, 16)] = v & 0xFFFF
                dic[slot, pl.ds(16 * k, 16)] = lax.shift_right_logical(v, 16)

        def drain_g(bt):
            pltpu.make_async_copy(
                u_hbm.at[pl.ds(0, CHP)], gb_v.at[bt], sem).wait()

        def drain_s():
            pltpu.make_async_copy(
                u_hbm.at[pl.ds(0, CHP)], acc_sh.at[pl.ds(0, CHP)],
                ssem).wait()

        unpack(0, 0)
        pltpu.async_copy(u_hbm.at[sic.at[0]], gb_v.at[0], sem)
        unpack(1, 1)
        pltpu.async_copy(u_hbm.at[sic.at[1]], gb_v.at[1], sem)

        def step(t, bt, start_next):
            drain_g(bt)
            pltpu.async_copy(
                gb_v.at[bt], acc_sh.at[dic.at[bt]], ssem, add=True)
            drain_s()
            if start_next:
                unpack(t + 2, bt)
                pltpu.async_copy(u_hbm.at[sic.at[bt]], gb_v.at[bt], sem)

        def body(p, carry):
            step(2 * p, 0, True)
            step(2 * p + 1, 1, True)
            return carry

        lax.fori_loop(0, NCHP // 2 - 1, body, 0)
        step(NCHP - 2, 0, False)
        step(NCHP - 1, 1, False)
        plsc.subcore_barrier()
        pltpu.sync_copy(acc_sh.at[pl.ds(s * RPS, RPS)],
                        out_hbm.at[c, pl.ds(s * RPS, RPS)])

    return agg_k(u, pk_pos, zeros_tab)


def _decode_call(z, pk_pos, pk_neg):
    D = z.shape[1]

    @functools.partial(
        pl.kernel,
        out_type=(jax.ShapeDtypeStruct((NW, EPWP), jnp.float32),
                  jax.ShapeDtypeStruct((NW, EPWP), jnp.float32)),
        compiler_params=pltpu.CompilerParams(needs_layout_passes=False),
        mesh=_mesh(),
        scratch_types=[
            pltpu.VMEM((NCHP, CHP), jnp.int32),
            pltpu.VMEM((2 * CHG,), jnp.int32),
            pltpu.VMEM((2 * CHG,), jnp.int32),
            pltpu.VMEM((2 * CHG, HID), jnp.float32),
            pltpu.VMEM((2 * CHG, HID), jnp.float32),
            pltpu.VMEM((2 * CHG,), jnp.float32),
            pltpu.VMEM((16 * 17,), jnp.float32),
            pltpu.SemaphoreType.DMA,
            pltpu.SemaphoreType.DMA,
        ],
    )
    def dec_k(z_hbm, pkp_h, pkn_h, po_h, no_h,
              pk_v, iac, ibc, a_v, b_v, sc_v, tp_v, sa, sb):
        c = lax.axis_index("c")
        s = lax.axis_index("s")
        wid = s * NC + c
        rows16 = lax.iota(jnp.int32, 16)

        def unpack_to(row, half, lo_ref, lo_slot, hi_ref, hi_slot):
            rvec = jnp.full((16,), row, jnp.int32)
            for k in range(CHG // 16):
                cvec = rows16 + (64 * half + 16 * k)
                v = plsc.load_gather(pk_v, [rvec, cvec])
                lo_ref[pl.ds(lo_slot * CHG + 16 * k, 16)] = v & 0xFFFF
                hi_ref[pl.ds(hi_slot * CHG + 16 * k, 16)] = (
                    lax.shift_right_logical(v, 16))

        def compute_chunk(bt):
            for q in range(CHG // 16):
                for l in range(16):
                    e = bt * CHG + 16 * q + l
                    part = (a_v[e, pl.ds(0, 16)] *
                            b_v[e, pl.ds(0, 16)])
                    for k in range(1, OUT_C // 16):
                        part = part + (a_v[e, pl.ds(16 * k, 16)] *
                                       b_v[e, pl.ds(16 * k, 16)])
                    tp_v[pl.ds(17 * l, 16)] = part
                res = plsc.load_gather(tp_v, [rows16 * 17])
                for k in range(1, 16):
                    res = res + plsc.load_gather(tp_v, [rows16 * 17 + k])
                sc_v[pl.ds(bt * CHG + 16 * q, 16)] = res

        def do_list(pk_hbm, o_h):
            pltpu.sync_copy(pk_hbm.at[wid], pk_v)

            def islc(ref, bb):
                return ref.at[pl.ds(bb * CHG, CHG)]

            def start_g(t, bt):
                pltpu.async_copy(z_hbm.at[islc(iac, bt)], islc(a_v, bt), sa)
                pltpu.async_copy(z_hbm.at[islc(ibc, bt)], islc(b_v, bt), sb)

            def drain_g(bt):
                pltpu.make_async_copy(
                    z_hbm.at[pl.ds(0, CHG)], islc(a_v, bt), sa).wait()
                pltpu.make_async_copy(
                    z_hbm.at[pl.ds(0, CHG)], islc(b_v, bt), sb).wait()

            unpack_to(0, 0, iac, 0, ibc, 0)
            start_g(0, 0)
            unpack_to(0, 1, iac, 1, ibc, 1)
            start_g(1, 1)

            def step(t, bt, start_next):
                drain_g(bt)
                compute_chunk(bt)
                if start_next:
                    unpack_to((t + 2) // 2, bt, iac, bt, ibc, bt)
                    start_g(t + 2, bt)

            def body(p, carry):
                step(2 * p, 0, True)
                step(2 * p + 1, 1, True)
                pltpu.sync_copy(
                    sc_v, o_h.at[wid, pl.ds(2 * p * CHG, 2 * CHG)])
                return carry

            lax.fori_loop(0, NCH // 2 - 1, body, 0)
            step(NCH - 2, 0, False)
            step(NCH - 1, 1, False)
            pltpu.sync_copy(
                sc_v, o_h.at[wid, pl.ds((NCH - 2) * CHG, 2 * CHG)])

        do_list(pkp_h, po_h)
        do_list(pkn_h, no_h)

    return dec_k(z, pk_pos, pk_neg)


def _dis_rows(dg_ref):
    deg = dg_ref[:, 0] + dg_ref[:, 1] + 1.0
    return lax.rsqrt(jnp.maximum(deg, 1e-12))


def _matmul(a, w_ref):
    return lax.dot_general(a, w_ref[...], (((1,), (0,)), ((), ())),
                           precision=lax.Precision.HIGHEST,
                           preferred_element_type=jnp.float32)


_BR = 2048


def _pre_call(degp, x, w1):
    def body(dg_ref, x_ref, w_ref, o_ref):
        dis = _dis_rows(dg_ref)[:, None]
        o_ref[...] = dis * _matmul(x_ref[...], w_ref)

    return pl.pallas_call(
        body,
        grid=(NPAD // _BR,),
        in_specs=[
            pl.BlockSpec((_BR, NC), lambda i: (i, 0)),
            pl.BlockSpec((_BR, IN_C), lambda i: (i, 0)),
            pl.BlockSpec((IN_C, HID), lambda i: (0, 0)),
        ],
        out_specs=pl.BlockSpec((_BR, HID), lambda i: (i, 0)),
        out_shape=jax.ShapeDtypeStruct((NPAD, HID), jnp.float32),
    )(degp, x, w1)


def _mid_call(degp, sp, u_prev, b, w_next):
    width = sp.shape[2]
    h = w_next.shape[1]

    def body(dg_ref, sp_ref, u_ref, b_ref, w_ref, o_ref):
        dis = _dis_rows(dg_ref)[:, None]
        stot = sp_ref[0] + sp_ref[1] + u_ref[...]
        zz = jnp.maximum(stot * dis + b_ref[...], 0.0)
        o_ref[...] = dis * _matmul(zz, w_ref)

    return pl.pallas_call(
        body,
        grid=(NPAD // _BR,),
        in_specs=[
            pl.BlockSpec((_BR, NC), lambda i: (i, 0)),
            pl.BlockSpec((NC, _BR, width), lambda i: (0, i, 0)),
            pl.BlockSpec((_BR, width), lambda i: (i, 0)),
            pl.BlockSpec((1, width), lambda i: (0, 0)),
            pl.BlockSpec((width, h), lambda i: (0, 0)),
        ],
        out_specs=pl.BlockSpec((_BR, h), lambda i: (i, 0)),
        out_shape=jax.ShapeDtypeStruct((NPAD, h), jnp.float32),
    )(degp, sp, u_prev, b, w_next)


def _fin_call(degp, sp, u_prev, b):
    width = sp.shape[2]

    def body(dg_ref, sp_ref, u_ref, b_ref, o_ref):
        dis = _dis_rows(dg_ref)[:, None]
        stot = sp_ref[0] + sp_ref[1] + u_ref[...]
        o_ref[...] = stot * dis + b_ref[...]

    return pl.pallas_call(
        body,
        grid=(NPAD // _BR,),
        in_specs=[
            pl.BlockSpec((_BR, NC), lambda i: (i, 0)),
            pl.BlockSpec((NC, _BR, width), lambda i: (0, i, 0)),
            pl.BlockSpec((_BR, width), lambda i: (i, 0)),
            pl.BlockSpec((1, width), lambda i: (0, 0)),
        ],
        out_specs=pl.BlockSpec((_BR, width), lambda i: (i, 0)),
        out_shape=jax.ShapeDtypeStruct((NPAD, width), jnp.float32),
    )(degp, sp, u_prev, b)


def _pack_edges(rows, cols):
    pad = N + (jnp.arange(EPWP - EPW, dtype=jnp.int32) % (NPAD - N))
    pad = jnp.broadcast_to(pad, (NW, EPWP - EPW))
    r2 = jnp.concatenate([rows.reshape(NW, EPW), pad], axis=1)
    c2 = jnp.concatenate([cols.reshape(NW, EPW), pad], axis=1)
    return (r2 + c2 * 65536).reshape(NW, NCHP, CHP)


def kernel(x, pos_edge_index, neg_edge_index, W1, b1, W2, b2, W3, b3):
    pk_pos = _pack_edges(pos_edge_index[0], pos_edge_index[1])
    pk_neg = _pack_edges(neg_edge_index[0], neg_edge_index[1])

    zeros_np = jnp.zeros((NP,), jnp.float32)
    zeros128 = jnp.zeros((NPAD, HID), jnp.float32)
    xp = jnp.pad(x, ((0, NPAD - N), (0, 0)))
    w3p = jnp.pad(W3, ((0, 0), (0, HID - OUT_C)))
    b3p = jnp.pad(b3, (0, HID - OUT_C))

    degp = _deg_call(pk_pos, zeros_np).T
    u1 = _pre_call(degp, xp, W1)
    s1 = _agg_call(u1, pk_pos, zeros128)
    u2 = _mid_call(degp, s1, u1, b1.reshape(1, HID), W2)
    s2 = _agg_call(u2, pk_pos, zeros128)
    u3 = _mid_call(degp, s2, u2, b2.reshape(1, HID), w3p)
    s3 = _agg_call(u3, pk_pos, zeros128)
    z3 = _fin_call(degp, s3, u3, b3p.reshape(1, HID))
    pos_p, neg_p = _decode_call(z3, pk_pos, pk_neg)
    pos_score = pos_p[:, :EPW].reshape(E)
    neg_score = neg_p[:, :EPW].reshape(E)
    return (pos_score, neg_score)
